# baseline (speedup 1.0000x reference)
import functools

import jax
import jax.numpy as jnp
from jax import lax
from jax.experimental import pallas as pl
from jax.experimental.pallas import tpu as pltpu
from jax.experimental.pallas import tpu_sc as plsc

_VOCAB = 1000000
_DIM = 64
_SCALE = 8.0
_NC = 2
_NS = 16
_NW = _NC * _NS
_BATCH = 4096
_SEQ = 200

_VC = 256
_TC = 128
_NFULL = _VOCAB // _VC
_KMAX = -(-_NFULL // _NW)


_RS = _DIM


def _transpose_chunk(slab, trans, width):
  rows = [(lax.iota(jnp.int32, 16) + 16 * g) * _RS for g in range(width // 16)]

  @plsc.parallel_loop(0, _DIM, unroll=2)
  def _(d):
    dful = jnp.full((16,), d, jnp.int32)
    for g in range(width // 16):
      vec = slab[d, pl.ds(16 * g, 16)]
      plsc.store_scatter(trans, [rows[g] + dful], vec)


def _a_body(tt_hbm, tail_hbm, tr_hbm, slab0, slab1, trn0, trn1, slab_t,
            trn_t, g0, g1, s0, s1, gt, st):
  slabs = [slab0, slab1]
  trns = [trn0, trn1]
  gsems = [g0, g1]
  ssems = [s0, s1]
  wid = lax.axis_index("s") * _NC + lax.axis_index("c")

  def chunk_of(k):
    return wid + k * _NW

  def slab_dst(b):
    return slabs[b]

  def wait_slab(b):
    pltpu.make_async_copy(tt_hbm.at[:, pl.ds(0, _VC)], slab_dst(b),
                          gsems[b]).wait()

  def wait_store(b):
    pltpu.make_async_copy(trns[b], tr_hbm.at[pl.ds(0, _VC * _RS)],
                          ssems[b]).wait()

  @pl.when(chunk_of(0) < _NFULL)
  def _():
    pltpu.async_copy(tt_hbm.at[:, pl.ds(chunk_of(0) * _VC, _VC)],
                     slab_dst(0), gsems[0])

  @pl.when(chunk_of(1) < _NFULL)
  def _():
    pltpu.async_copy(tt_hbm.at[:, pl.ds(chunk_of(1) * _VC, _VC)],
                     slab_dst(1), gsems[1])

  def step(o, carry):
    for b in range(2):
      k = o * 2 + b
      c = chunk_of(k)

      @pl.when(c < _NFULL)
      def _(b=b, k=k, c=c):
        wait_slab(b)

        @pl.when(k >= 2)
        def _():
          wait_store(b)
        _transpose_chunk(slabs[b], trns[b], _VC)

        @pl.when(chunk_of(k + 2) < _NFULL)
        def _():
          pltpu.async_copy(tt_hbm.at[:, pl.ds(chunk_of(k + 2) * _VC, _VC)],
                           slab_dst(b), gsems[b])
        pltpu.async_copy(trns[b], tr_hbm.at[pl.ds(c * _VC * _RS, _VC * _RS)],
                         ssems[b])
    return carry

  lax.fori_loop(0, (_KMAX + 1) // 2, step, 0)

  @pl.when(wid == _NW - 1)
  def _():
    pltpu.async_copy(tail_hbm, slab_t, gt)
    pltpu.make_async_copy(tail_hbm, slab_t, gt).wait()
    _transpose_chunk(slab_t, trn_t, _TC)
    base = (_VOCAB - _TC) * _RS
    pltpu.async_copy(trn_t, tr_hbm.at[pl.ds(base, _TC * _RS)], st)
    pltpu.make_async_copy(trn_t, tr_hbm.at[pl.ds(0, _TC * _RS)], st).wait()

  wait_store(0)
  wait_store(1)


_a_call = pl.kernel(
    _a_body,
    out_type=jax.ShapeDtypeStruct((_VOCAB * _RS,), jnp.float32),
    mesh=plsc.VectorSubcoreMesh(core_axis_name="c", subcore_axis_name="s"),
    scratch_types=(
        [pltpu.VMEM((_DIM, _VC), jnp.float32) for _ in range(2)]
        + [pltpu.VMEM((_VC * _RS,), jnp.float32) for _ in range(2)]
        + [pltpu.VMEM((_DIM, _TC), jnp.float32),
           pltpu.VMEM((_TC * _RS,), jnp.float32)]
        + [pltpu.SemaphoreType.DMA for _ in range(6)]
    ),
    compiler_params=pltpu.CompilerParams(use_tc_tiling_on_sc=True,
                                         needs_layout_passes=False),
)

_NB = 4
_NO = 2
_BPW = _BATCH // _NW
_OST = _BPW + 1


def _b_body(xt_hbm, tr_hbm, out_hbm, idx_v, r0, r1, r2, r3, o0, o1,
            g0, g1, g2, g3, s0, s1):
  rbufs = [r0, r1, r2, r3]
  obufs = [o0, o1]
  gsems = [g0, g1, g2, g3]
  osems = [s0, s1]
  wid = lax.axis_index("s") * _NC + lax.axis_index("c")
  b0 = wid * _BPW

  pltpu.sync_copy(xt_hbm.at[:, pl.ds(b0, _BPW)], idx_v)

  def gather(t, b):
    pltpu.async_copy(tr_hbm.at[idx_v.at[t]], rbufs[b], gsems[b])

  def wait_gather(b):
    pltpu.make_async_copy(tr_hbm.at[idx_v.at[0]], rbufs[b],
                          gsems[b]).wait()

  def ostore(t, o):
    pltpu.async_copy(obufs[o].at[:, pl.ds(0, _BPW)],
                     out_hbm.at[t, :, pl.ds(b0, _BPW)], osems[o])

  def wait_ostore(o):
    pltpu.make_async_copy(obufs[o].at[:, pl.ds(0, _BPW)],
                          out_hbm.at[0, :, pl.ds(0, _BPW)], osems[o]).wait()

  rows = [lax.iota(jnp.int32, 16) + 16 * c for c in range(_DIM // 16)]

  def transpose_block(b, o):
    @plsc.parallel_loop(0, _BPW, unroll=2)
    def _(j):
      jful = jnp.full((16,), j, jnp.int32)
      for c in range(_DIM // 16):
        vec = rbufs[b][j, pl.ds(16 * c, 16)]
        plsc.store_scatter(obufs[o], [rows[c], jful], vec * _SCALE)

  for b in range(_NB):
    gather(b, b)

  def step(o_, carry):
    for v in range(_NB):
      t = o_ * _NB + v
      b = v
      u = v % _NO
      wait_gather(b)

      @pl.when(t >= _NO)
      def _():
        wait_ostore(u)
      transpose_block(b, u)

      @pl.when(t < _SEQ - _NB)
      def _():
        gather(t + _NB, b)
      ostore(t, u)
    return carry

  lax.fori_loop(0, _SEQ // _NB, step, 0)

  for o in range(_NO):
    wait_ostore(o)


_b_call = pl.kernel(
    _b_body,
    out_type=jax.ShapeDtypeStruct((_SEQ, _DIM, _BATCH), jnp.float32),
    mesh=plsc.VectorSubcoreMesh(core_axis_name="c", subcore_axis_name="s"),
    scratch_types=(
        [pltpu.VMEM((_SEQ, _BPW), jnp.int32)]
        + [pltpu.VMEM((_BPW, _RS), jnp.float32) for _ in range(_NB)]
        + [pltpu.VMEM((_DIM, _OST), jnp.float32) for _ in range(_NO)]
        + [pltpu.SemaphoreType.DMA for _ in range(_NB + _NO)]
    ),
    compiler_params=pltpu.CompilerParams(use_tc_tiling_on_sc=False,
                                         needs_layout_passes=False),
)


@jax.jit
def kernel(x, table):
  xt = x.astype(jnp.int32).T
  tt = table.T
  tail = lax.slice(tt, (0, _VOCAB - _TC), (_DIM, _VOCAB))
  tr = _a_call(tt, tail).reshape(_VOCAB, _RS)
  out_t = _b_call(xt, tr)
  return out_t.transpose(2, 0, 1)


# speedup vs baseline: 1.0635x; 1.0635x over previous
import functools

import jax
import jax.numpy as jnp
from jax import lax
from jax.experimental import pallas as pl
from jax.experimental.pallas import tpu as pltpu
from jax.experimental.pallas import tpu_sc as plsc

_VOCAB = 1000000
_DIM = 64
_SCALE = 8.0
_NC = 2
_NS = 16
_NW = _NC * _NS
_BATCH = 4096
_SEQ = 200

_VC = 256
_TC = 128
_NFULL = _VOCAB // _VC
_KMAX = -(-_NFULL // _NW)


_RS = _DIM


def _transpose_chunk(slab, trans, width):
  rows = [lax.iota(jnp.int32, 16) + 16 * f for f in range(_DIM // 16)]

  @plsc.parallel_loop(0, width, unroll=4)
  def _(v):
    col = jnp.full((16,), v, jnp.int32)
    for f in range(_DIM // 16):
      vec = plsc.load_gather(slab, [rows[f], col])
      trans[pl.ds(v * _DIM + 16 * f, 16)] = vec


def _a_body(tt_hbm, tail_hbm, tr_hbm, slab0, slab1, trn0, trn1, slab_t,
            trn_t, g0, g1, s0, s1, gt, st):
  slabs = [slab0, slab1]
  trns = [trn0, trn1]
  gsems = [g0, g1]
  ssems = [s0, s1]
  wid = lax.axis_index("s") * _NC + lax.axis_index("c")

  def chunk_of(k):
    return wid + k * _NW

  def slab_dst(b):
    return slabs[b]

  def wait_slab(b):
    pltpu.make_async_copy(tt_hbm.at[:, pl.ds(0, _VC)], slab_dst(b),
                          gsems[b]).wait()

  def wait_store(b):
    pltpu.make_async_copy(trns[b], tr_hbm.at[pl.ds(0, _VC * _RS)],
                          ssems[b]).wait()

  @pl.when(chunk_of(0) < _NFULL)
  def _():
    pltpu.async_copy(tt_hbm.at[:, pl.ds(chunk_of(0) * _VC, _VC)],
                     slab_dst(0), gsems[0])

  @pl.when(chunk_of(1) < _NFULL)
  def _():
    pltpu.async_copy(tt_hbm.at[:, pl.ds(chunk_of(1) * _VC, _VC)],
                     slab_dst(1), gsems[1])

  def step(o, carry):
    for b in range(2):
      k = o * 2 + b
      c = chunk_of(k)

      @pl.when(c < _NFULL)
      def _(b=b, k=k, c=c):
        wait_slab(b)

        @pl.when(k >= 2)
        def _():
          wait_store(b)
        _transpose_chunk(slabs[b], trns[b], _VC)

        @pl.when(chunk_of(k + 2) < _NFULL)
        def _():
          pltpu.async_copy(tt_hbm.at[:, pl.ds(chunk_of(k + 2) * _VC, _VC)],
                           slab_dst(b), gsems[b])
        pltpu.async_copy(trns[b], tr_hbm.at[pl.ds(c * _VC * _RS, _VC * _RS)],
                         ssems[b])
    return carry

  lax.fori_loop(0, (_KMAX + 1) // 2, step, 0)

  @pl.when(wid == _NW - 1)
  def _():
    pltpu.async_copy(tail_hbm, slab_t, gt)
    pltpu.make_async_copy(tail_hbm, slab_t, gt).wait()
    _transpose_chunk(slab_t, trn_t, _TC)
    base = (_VOCAB - _TC) * _RS
    pltpu.async_copy(trn_t, tr_hbm.at[pl.ds(base, _TC * _RS)], st)
    pltpu.make_async_copy(trn_t, tr_hbm.at[pl.ds(0, _TC * _RS)], st).wait()

  wait_store(0)
  wait_store(1)


_a_call = pl.kernel(
    _a_body,
    out_type=jax.ShapeDtypeStruct((_VOCAB * _RS,), jnp.float32),
    mesh=plsc.VectorSubcoreMesh(core_axis_name="c", subcore_axis_name="s"),
    scratch_types=(
        [pltpu.VMEM((_DIM, _VC), jnp.float32) for _ in range(2)]
        + [pltpu.VMEM((_VC * _RS,), jnp.float32) for _ in range(2)]
        + [pltpu.VMEM((_DIM, _TC), jnp.float32),
           pltpu.VMEM((_TC * _RS,), jnp.float32)]
        + [pltpu.SemaphoreType.DMA for _ in range(6)]
    ),
    compiler_params=pltpu.CompilerParams(use_tc_tiling_on_sc=True,
                                         needs_layout_passes=False),
)

_NB = 4
_NO = 2
_BPW = _BATCH // _NW
_OST = _BPW + 1


def _b_body(xt_hbm, tr_hbm, out_hbm, idx_v, r0, r1, r2, r3, o0, o1,
            g0, g1, g2, g3, s0, s1):
  rbufs = [r0, r1, r2, r3]
  obufs = [o0, o1]
  gsems = [g0, g1, g2, g3]
  osems = [s0, s1]
  wid = lax.axis_index("s") * _NC + lax.axis_index("c")
  b0 = wid * _BPW

  pltpu.sync_copy(xt_hbm.at[:, pl.ds(b0, _BPW)], idx_v)

  def gather(t, b):
    pltpu.async_copy(tr_hbm.at[idx_v.at[t]], rbufs[b], gsems[b])

  def wait_gather(b):
    pltpu.make_async_copy(tr_hbm.at[idx_v.at[0]], rbufs[b],
                          gsems[b]).wait()

  def ostore(t, o):
    pltpu.async_copy(obufs[o].at[:, pl.ds(0, _BPW)],
                     out_hbm.at[t, :, pl.ds(b0, _BPW)], osems[o])

  def wait_ostore(o):
    pltpu.make_async_copy(obufs[o].at[:, pl.ds(0, _BPW)],
                          out_hbm.at[0, :, pl.ds(0, _BPW)], osems[o]).wait()

  rows = [lax.iota(jnp.int32, 16) + 16 * c for c in range(_DIM // 16)]

  def transpose_block(b, o):
    @plsc.parallel_loop(0, _BPW, unroll=2)
    def _(j):
      jful = jnp.full((16,), j, jnp.int32)
      for c in range(_DIM // 16):
        vec = rbufs[b][j, pl.ds(16 * c, 16)]
        plsc.store_scatter(obufs[o], [rows[c], jful], vec * _SCALE)

  for b in range(_NB):
    gather(b, b)

  def step(o_, carry):
    for v in range(_NB):
      t = o_ * _NB + v
      b = v
      u = v % _NO
      wait_gather(b)

      @pl.when(t >= _NO)
      def _():
        wait_ostore(u)
      transpose_block(b, u)

      @pl.when(t < _SEQ - _NB)
      def _():
        gather(t + _NB, b)
      ostore(t, u)
    return carry

  lax.fori_loop(0, _SEQ // _NB, step, 0)

  for o in range(_NO):
    wait_ostore(o)


_b_call = pl.kernel(
    _b_body,
    out_type=jax.ShapeDtypeStruct((_SEQ, _DIM, _BATCH), jnp.float32),
    mesh=plsc.VectorSubcoreMesh(core_axis_name="c", subcore_axis_name="s"),
    scratch_types=(
        [pltpu.VMEM((_SEQ, _BPW), jnp.int32)]
        + [pltpu.VMEM((_BPW, _RS), jnp.float32) for _ in range(_NB)]
        + [pltpu.VMEM((_DIM, _OST), jnp.float32) for _ in range(_NO)]
        + [pltpu.SemaphoreType.DMA for _ in range(_NB + _NO)]
    ),
    compiler_params=pltpu.CompilerParams(use_tc_tiling_on_sc=False,
                                         needs_layout_passes=False),
)


@jax.jit
def kernel(x, table):
  xt = x.astype(jnp.int32).T
  tt = table.T
  tail = lax.slice(tt, (0, _VOCAB - _TC), (_DIM, _VOCAB))
  tr = _a_call(tt, tail).reshape(_VOCAB, _RS)
  out_t = _b_call(xt, tr)
  return out_t.transpose(2, 0, 1)
